# baseline (speedup 1.0000x reference)
import functools

import jax
import jax.numpy as jnp
from jax import lax
from jax.experimental import pallas as pl
from jax.experimental.pallas import tpu as pltpu
from jax.experimental.pallas import tpu_sc as plsc

N = 32768
C = 512
NC = 2
NS = 16
NW = NC * NS
R = N // NW
L = 16
GCHUNK = 128
NG = R // GCHUNK


@functools.partial(
    pl.kernel,
    out_type=jax.ShapeDtypeStruct((N,), jnp.float32),
    mesh=plsc.VectorSubcoreMesh(core_axis_name="c", subcore_axis_name="s"),
    compiler_params=pltpu.CompilerParams(needs_layout_passes=False),
    scratch_types=[
        pltpu.VMEM((R,), jnp.int32),
        pltpu.VMEM((R,), jnp.int32),
        pltpu.VMEM((R,), jnp.float32),
        pltpu.SemaphoreType.DMA,
        pltpu.SemaphoreType.DMA,
    ],
)
def _sc_gather(table_hbm, chord_hbm, out_hbm, chord_v, idx_v, picked_v,
               sem, sem2):
    cid = lax.axis_index("c")
    sid = lax.axis_index("s")
    wid = sid * NC + cid
    base = wid * R

    pltpu.sync_copy(chord_hbm.at[pl.ds(base, R)], chord_v)

    iota = lax.iota(jnp.int32, L)
    lanes = ((iota >> 3) << 12) + ((iota & 7) << 7)

    def idx_body(t, _):
        off = pl.multiple_of(t * L, L)
        c = chord_v[pl.ds(off, L)]
        rb = (base + t * L) >> 3
        idx_v[pl.ds(off, L)] = (
            ((rb << 12) + lanes) + (((c >> 7) << 10) + (c & 127))
        )
        return 0

    lax.fori_loop(0, R // L, idx_body, 0, unroll=2)

    pltpu.async_copy(table_hbm.at[idx_v], picked_v, sem).wait()
    pltpu.sync_copy(picked_v, out_hbm.at[pl.ds(base, R)])


def _masked_mean_body(picked_ref, mask_ref, out_ref):
    p = picked_ref[...]
    m = mask_ref[...].astype(jnp.float32)
    s = jnp.sum(p * m)
    c = jnp.sum(m)
    out_ref[0, 0] = -s / c


_masked_mean = pl.pallas_call(
    _masked_mean_body,
    out_shape=jax.ShapeDtypeStruct((1, 1), jnp.float32),
    out_specs=pl.BlockSpec(memory_space=pltpu.SMEM),
)


def kernel(predict_chord, chord, mask):
    table = (
        predict_chord.reshape(N // 8, 8, C // 128, 128)
        .transpose(0, 2, 1, 3)
        .reshape(-1)
    )
    picked = _sc_gather(table, chord)
    loss = _masked_mean(picked.reshape(N // 128, 128),
                        mask.reshape(N // 128, 128))
    return loss[0, 0]


# speedup vs baseline: 1.0001x; 1.0001x over previous
import functools

import jax
import jax.numpy as jnp
from jax import lax
from jax.experimental import pallas as pl
from jax.experimental.pallas import tpu as pltpu
from jax.experimental.pallas import tpu_sc as plsc

N = 32768
C = 512
NC = 2
NS = 16
NW = NC * NS
R = N // NW
L = 16
GCHUNK = 128
NG = R // GCHUNK


@functools.partial(
    pl.kernel,
    out_type=jax.ShapeDtypeStruct((N,), jnp.float32),
    mesh=plsc.VectorSubcoreMesh(core_axis_name="c", subcore_axis_name="s"),
    compiler_params=pltpu.CompilerParams(
        needs_layout_passes=False, skip_device_barrier=True),
    scratch_types=[
        pltpu.VMEM((R,), jnp.int32),
        pltpu.VMEM((R,), jnp.int32),
        pltpu.VMEM((R,), jnp.float32),
        pltpu.SemaphoreType.DMA,
        pltpu.SemaphoreType.DMA,
    ],
)
def _sc_gather(table_hbm, chord_hbm, out_hbm, chord_v, idx_v, picked_v,
               sem, sem2):
    cid = lax.axis_index("c")
    sid = lax.axis_index("s")
    wid = sid * NC + cid
    base = wid * R

    pltpu.sync_copy(chord_hbm.at[pl.ds(base, R)], chord_v)

    iota = lax.iota(jnp.int32, L)
    lanes = ((iota >> 3) << 12) + ((iota & 7) << 7)

    def idx_body(t, _):
        off = pl.multiple_of(t * L, L)
        c = chord_v[pl.ds(off, L)]
        rb = (base + t * L) >> 3
        idx_v[pl.ds(off, L)] = (
            ((rb << 12) + lanes) + (((c >> 7) << 10) + (c & 127))
        )
        return 0

    lax.fori_loop(0, R // L, idx_body, 0, unroll=2)

    pltpu.async_copy(table_hbm.at[idx_v], picked_v, sem).wait()
    pltpu.sync_copy(picked_v, out_hbm.at[pl.ds(base, R)])


def _masked_mean_body(picked_ref, mask_ref, out_ref):
    p = picked_ref[...]
    m = mask_ref[...].astype(jnp.float32)
    s = jnp.sum(p * m)
    c = jnp.sum(m)
    out_ref[0, 0] = -s / c


_masked_mean = pl.pallas_call(
    _masked_mean_body,
    out_shape=jax.ShapeDtypeStruct((1, 1), jnp.float32),
    out_specs=pl.BlockSpec(memory_space=pltpu.SMEM),
)


def kernel(predict_chord, chord, mask):
    table = (
        predict_chord.reshape(N // 8, 8, C // 128, 128)
        .transpose(0, 2, 1, 3)
        .reshape(-1)
    )
    picked = _sc_gather(table, chord)
    loss = _masked_mean(picked.reshape(N // 128, 128),
                        mask.reshape(N // 128, 128))
    return loss[0, 0]


# speedup vs baseline: 1.0067x; 1.0066x over previous
import functools

import jax
import jax.numpy as jnp
from jax import lax
from jax.experimental import pallas as pl
from jax.experimental.pallas import tpu as pltpu
from jax.experimental.pallas import tpu_sc as plsc

N = 32768
C = 512
NC = 2
NS = 16
NW = NC * NS
R = N // NW
L = 16
GCHUNK = 128
NG = R // GCHUNK


@functools.partial(
    pl.kernel,
    out_type=jax.ShapeDtypeStruct((N,), jnp.float32),
    mesh=plsc.VectorSubcoreMesh(core_axis_name="c", subcore_axis_name="s"),
    compiler_params=pltpu.CompilerParams(needs_layout_passes=False),
    scratch_types=[
        pltpu.VMEM((R,), jnp.int32),
        pltpu.VMEM((R,), jnp.int32),
        pltpu.VMEM((R,), jnp.float32),
        pltpu.SemaphoreType.DMA,
        pltpu.SemaphoreType.DMA,
    ],
)
def _sc_gather(table_hbm, chord_hbm, out_hbm, chord_v, idx_v, picked_v,
               sem, sem2):
    cid = lax.axis_index("c")
    sid = lax.axis_index("s")
    wid = sid * NC + cid
    base = wid * R

    H = R // 2
    chord_cp = [
        pltpu.async_copy(
            chord_hbm.at[pl.ds(base + h * H, H)],
            chord_v.at[pl.ds(h * H, H)], sem2)
        for h in range(2)
    ]

    iota = lax.iota(jnp.int32, L)
    lanes = ((iota >> 3) << 12) + ((iota & 7) << 7)

    def idx_body(t, _):
        off = pl.multiple_of(t * L, L)
        c = chord_v[pl.ds(off, L)]
        rb = (base + t * L) >> 3
        idx_v[pl.ds(off, L)] = (
            ((rb << 12) + lanes) + (((c >> 7) << 10) + (c & 127))
        )
        return 0

    gathers = []
    for h in range(2):
        chord_cp[h].wait()
        lax.fori_loop(h * (H // L), (h + 1) * (H // L), idx_body, 0, unroll=2)
        gathers.append(
            pltpu.async_copy(
                table_hbm.at[idx_v.at[pl.ds(h * H, H)]],
                picked_v.at[pl.ds(h * H, H)], sem))
    writes = []
    for h in range(2):
        gathers[h].wait()
        writes.append(
            pltpu.async_copy(
                picked_v.at[pl.ds(h * H, H)],
                out_hbm.at[pl.ds(base + h * H, H)], sem2))
    for w in writes:
        w.wait()


def _masked_mean_body(picked_ref, mask_ref, out_ref):
    p = picked_ref[...]
    m = mask_ref[...].astype(jnp.float32)
    s = jnp.sum(p * m)
    c = jnp.sum(m)
    out_ref[0, 0] = -s / c


_masked_mean = pl.pallas_call(
    _masked_mean_body,
    out_shape=jax.ShapeDtypeStruct((1, 1), jnp.float32),
    out_specs=pl.BlockSpec(memory_space=pltpu.SMEM),
)


def kernel(predict_chord, chord, mask):
    table = (
        predict_chord.reshape(N // 8, 8, C // 128, 128)
        .transpose(0, 2, 1, 3)
        .reshape(-1)
    )
    picked = _sc_gather(table, chord)
    loss = _masked_mean(picked.reshape(N // 128, 128),
                        mask.reshape(N // 128, 128))
    return loss[0, 0]
